# baseline (speedup 1.0000x reference)
import functools

import jax
import jax.numpy as jnp
from jax import lax
from jax.experimental import pallas as pl
from jax.experimental.pallas import tpu as pltpu
from jax.experimental.pallas import tpu_sc as plsc

VOCAB = 1000
EMBED = 128
BATCH = 1024
SEQ = 20
SEQ_PAD = 24

N_SLICE = 2
B_SLICE = BATCH // N_SLICE
TOK_SLICE = B_SLICE * SEQ_PAD

B_BLK = 64
BLK_PER_SLICE = B_SLICE // B_BLK


@functools.lru_cache(maxsize=1)
def _make_gather_kernel():
    info = plsc.get_sparse_core_info()
    nw = info.num_cores * info.num_subcores
    per_w = TOK_SLICE // nw
    chunk = 128
    n_chunks = per_w // chunk
    mesh = plsc.VectorSubcoreMesh(core_axis_name="c", subcore_axis_name="s")

    @functools.partial(
        pl.kernel,
        out_type=jax.ShapeDtypeStruct((TOK_SLICE, EMBED), jnp.float32),
        mesh=mesh,
        scratch_types=[
            pltpu.VMEM((per_w,), jnp.int32),
            pltpu.VMEM((per_w, EMBED), jnp.float32),
            pltpu.SemaphoreType.DMA,
        ],
    )
    def gather_k(table_hbm, idx_hbm, out_hbm, idx_v, rows_v, sem):
        wid = lax.axis_index("s") * info.num_cores + lax.axis_index("c")
        base = wid * per_w
        pltpu.sync_copy(idx_hbm.at[pl.ds(base, per_w)], idx_v)
        handles = [
            pltpu.async_copy(
                table_hbm.at[idx_v.at[pl.ds(c * chunk, chunk)]],
                rows_v.at[pl.ds(c * chunk, chunk)],
                sem,
            )
            for c in range(n_chunks)
        ]
        for h in handles:
            h.wait()
        pltpu.sync_copy(rows_v, out_hbm.at[pl.ds(base, per_w)])

    return gather_k


def _proj_body(x_ref, w_ref, b_ref, alias_ref, o_ref):
    del alias_ref
    res = (
        jnp.dot(
            x_ref[...].astype(jnp.bfloat16),
            w_ref[...].astype(jnp.bfloat16),
            preferred_element_type=jnp.float32,
        )
        + b_ref[...]
    )
    o_ref[...] = res.reshape(B_BLK, SEQ_PAD, VOCAB)[:, :SEQ, :]


def _proj_slice(emb, w, b2d, out_prev, slice_idx):
    blk0 = slice_idx * BLK_PER_SLICE
    return pl.pallas_call(
        _proj_body,
        grid=(BLK_PER_SLICE,),
        in_specs=[
            pl.BlockSpec((B_BLK * SEQ_PAD, EMBED), lambda i: (i, 0)),
            pl.BlockSpec((EMBED, VOCAB), lambda i: (0, 0)),
            pl.BlockSpec((1, VOCAB), lambda i: (0, 0)),
            pl.BlockSpec(memory_space=pl.ANY),
        ],
        out_specs=pl.BlockSpec(
            (B_BLK, SEQ, VOCAB), lambda i, b0=blk0: (b0 + i, 0, 0)
        ),
        out_shape=jax.ShapeDtypeStruct((BATCH, SEQ, VOCAB), jnp.float32),
        input_output_aliases={3: 0},
    )(emb, w, b2d, out_prev)


def kernel(input_ids, embed_table, dense_kernel, dense_bias):
    ids32 = input_ids.astype(jnp.int32)
    ids_pad = jnp.concatenate([ids32, ids32[:, : SEQ_PAD - SEQ]], axis=1)
    gather = _make_gather_kernel()
    embs = [
        gather(
            embed_table,
            ids_pad[k * B_SLICE : (k + 1) * B_SLICE].reshape(TOK_SLICE),
        )
        for k in range(N_SLICE)
    ]
    b2d = dense_bias.reshape(1, VOCAB)
    out = jnp.empty((BATCH, SEQ, VOCAB), jnp.float32)
    for k in range(N_SLICE):
        out = _proj_slice(embs[k], dense_kernel, b2d, out, k)
    return out


# speedup vs baseline: 1.2040x; 1.2040x over previous
import functools

import jax
import jax.numpy as jnp
from jax import lax
from jax.experimental import pallas as pl
from jax.experimental.pallas import tpu as pltpu
from jax.experimental.pallas import tpu_sc as plsc

VOCAB = 1000
EMBED = 128
BATCH = 1024
SEQ = 20
SEQ_PAD = 24

N_SLICE = 2
B_SLICE = BATCH // N_SLICE
TOK_SLICE = B_SLICE * SEQ_PAD

B_BLK = 64
BLK_PER_SLICE = B_SLICE // B_BLK


@functools.lru_cache(maxsize=1)
def _make_gather_kernel():
    info = plsc.get_sparse_core_info()
    nw = info.num_cores * info.num_subcores
    per_w = TOK_SLICE // nw
    chunk = 128
    n_chunks = per_w // chunk
    mesh = plsc.VectorSubcoreMesh(core_axis_name="c", subcore_axis_name="s")

    @functools.partial(
        pl.kernel,
        out_type=jax.ShapeDtypeStruct((TOK_SLICE, EMBED), jnp.float32),
        mesh=mesh,
        scratch_types=[
            pltpu.VMEM((per_w,), jnp.int32),
            pltpu.VMEM((per_w, EMBED), jnp.float32),
            pltpu.SemaphoreType.DMA,
        ],
    )
    def gather_k(table_hbm, idx_hbm, out_hbm, idx_v, rows_v, sem):
        wid = lax.axis_index("s") * info.num_cores + lax.axis_index("c")
        base = wid * per_w
        pltpu.sync_copy(idx_hbm.at[pl.ds(base, per_w)], idx_v)
        handles = [
            pltpu.async_copy(
                table_hbm.at[idx_v.at[pl.ds(c * chunk, chunk)]],
                rows_v.at[pl.ds(c * chunk, chunk)],
                sem,
            )
            for c in range(n_chunks)
        ]
        for h in handles:
            h.wait()
        pltpu.sync_copy(rows_v, out_hbm.at[pl.ds(base, per_w)])

    return gather_k


def _proj_body(x_ref, w_ref, b_ref, o_ref):
    res = (
        jnp.dot(
            x_ref[...].astype(jnp.bfloat16),
            w_ref[...].astype(jnp.bfloat16),
            preferred_element_type=jnp.float32,
        )
        + b_ref[...]
    )
    o_ref[...] = res.reshape(B_BLK, SEQ_PAD, VOCAB)[:, :SEQ, :]


def _proj_slice(emb, w, b2d, out_prev, slice_idx):
    blk0 = slice_idx * BLK_PER_SLICE
    in_specs = [
        pl.BlockSpec((B_BLK * SEQ_PAD, EMBED), lambda i: (i, 0)),
        pl.BlockSpec((EMBED, VOCAB), lambda i: (0, 0)),
        pl.BlockSpec((1, VOCAB), lambda i: (0, 0)),
    ]
    args = [emb, w, b2d]
    aliases = {}
    body = _proj_body
    if out_prev is not None:
        in_specs.append(pl.BlockSpec(memory_space=pl.ANY))
        args.append(out_prev)
        aliases = {3: 0}

        def body(x_ref, w_ref, b_ref, alias_ref, o_ref):
            del alias_ref
            _proj_body(x_ref, w_ref, b_ref, o_ref)

    return pl.pallas_call(
        body,
        grid=(BLK_PER_SLICE,),
        in_specs=in_specs,
        out_specs=pl.BlockSpec(
            (B_BLK, SEQ, VOCAB), lambda i, b0=blk0: (b0 + i, 0, 0)
        ),
        out_shape=jax.ShapeDtypeStruct((BATCH, SEQ, VOCAB), jnp.float32),
        input_output_aliases=aliases,
    )(*args)


def kernel(input_ids, embed_table, dense_kernel, dense_bias):
    ids32 = input_ids.astype(jnp.int32)
    ids_pad = jnp.concatenate([ids32, ids32[:, : SEQ_PAD - SEQ]], axis=1)
    gather = _make_gather_kernel()
    embs = [
        gather(
            embed_table,
            ids_pad[k * B_SLICE : (k + 1) * B_SLICE].reshape(TOK_SLICE),
        )
        for k in range(N_SLICE)
    ]
    b2d = dense_bias.reshape(1, VOCAB)
    out = None
    for k in range(N_SLICE):
        out = _proj_slice(embs[k], dense_kernel, b2d, out, k)
    return out


# speedup vs baseline: 1.2374x; 1.0278x over previous
import functools

import jax
import jax.numpy as jnp
from jax import lax
from jax.experimental import pallas as pl
from jax.experimental.pallas import tpu as pltpu
from jax.experimental.pallas import tpu_sc as plsc

VOCAB = 1000
EMBED = 128
BATCH = 1024
SEQ = 20
SEQ_PAD = 24
NTOK_PAD = BATCH * SEQ_PAD


@functools.lru_cache(maxsize=1)
def _make_gather_kernel():
    info = plsc.get_sparse_core_info()
    nw = info.num_cores * info.num_subcores
    per_w = NTOK_PAD // nw
    chunk = 128
    n_chunks = per_w // chunk
    mesh = plsc.VectorSubcoreMesh(core_axis_name="c", subcore_axis_name="s")

    @functools.partial(
        pl.kernel,
        out_type=jax.ShapeDtypeStruct((NTOK_PAD, EMBED), jnp.float32),
        mesh=mesh,
        scratch_types=[
            pltpu.VMEM((per_w,), jnp.int32),
            pltpu.VMEM((per_w, EMBED), jnp.float32),
            pltpu.SemaphoreType.DMA,
        ],
    )
    def gather_k(table_hbm, idx_hbm, out_hbm, idx_v, rows_v, sem):
        wid = lax.axis_index("s") * info.num_cores + lax.axis_index("c")
        base = wid * per_w
        pltpu.sync_copy(idx_hbm.at[pl.ds(base, per_w)], idx_v)
        handles = [
            pltpu.async_copy(
                table_hbm.at[idx_v.at[pl.ds(c * chunk, chunk)]],
                rows_v.at[pl.ds(c * chunk, chunk)],
                sem,
            )
            for c in range(n_chunks)
        ]
        for h in handles:
            h.wait()
        pltpu.sync_copy(rows_v, out_hbm.at[pl.ds(base, per_w)])

    return gather_k


B_BLK = 128


def _proj_body(x_ref, w_ref, b_ref, o_ref):
    res = (
        jnp.dot(
            x_ref[...].astype(jnp.bfloat16),
            w_ref[...].astype(jnp.bfloat16),
            preferred_element_type=jnp.float32,
        )
        + b_ref[...]
    )
    o_ref[...] = res.reshape(B_BLK, SEQ_PAD, VOCAB)[:, :SEQ, :]


def kernel(input_ids, embed_table, dense_kernel, dense_bias):
    ids32 = input_ids.astype(jnp.int32)
    ids_pad = jnp.concatenate([ids32, ids32[:, : SEQ_PAD - SEQ]], axis=1)
    emb = _make_gather_kernel()(embed_table, ids_pad.reshape(NTOK_PAD))
    out = pl.pallas_call(
        _proj_body,
        grid=(BATCH // B_BLK,),
        in_specs=[
            pl.BlockSpec((B_BLK * SEQ_PAD, EMBED), lambda i: (i, 0)),
            pl.BlockSpec((EMBED, VOCAB), lambda i: (0, 0)),
            pl.BlockSpec((1, VOCAB), lambda i: (0, 0)),
        ],
        out_specs=pl.BlockSpec((B_BLK, SEQ, VOCAB), lambda i: (i, 0, 0)),
        out_shape=jax.ShapeDtypeStruct((BATCH, SEQ, VOCAB), jnp.float32),
    )(emb, dense_kernel, dense_bias.reshape(1, VOCAB))
    return out
